# baseline (speedup 1.0000x reference)
import functools

import jax
import jax.numpy as jnp
from jax import lax
from jax.experimental import pallas as pl
from jax.experimental.pallas import tpu as pltpu
from jax.experimental.pallas import tpu_sc as plsc


@functools.lru_cache(maxsize=None)
def _make_gather(V, D, B):
  info = plsc.get_sparse_core_info()
  NC, NS = info.num_cores, info.num_subcores
  NW = NC * NS
  assert B % NW == 0
  b_per_w = B // NW
  K = 8
  NBUF = 4
  assert b_per_w % (NBUF * K) == 0
  n_chunks = b_per_w // K
  mesh = plsc.VectorSubcoreMesh(core_axis_name="c", subcore_axis_name="s")

  @functools.partial(
      pl.kernel,
      mesh=mesh,
      out_type=jax.ShapeDtypeStruct((B, D), jnp.float32),
      scratch_types=[
          pltpu.VMEM((b_per_w,), jnp.int32),
      ]
      + [pltpu.VMEM((K, D), jnp.float32) for _ in range(NBUF)]
      + [pltpu.SemaphoreType.DMA for _ in range(2 * NBUF)],
  )
  def k(idx_hbm, table_hbm, out_hbm, idx_v, *bufs_and_sems):
    bufs = bufs_and_sems[:NBUF]
    gsem = bufs_and_sems[NBUF : 2 * NBUF]
    wsem = bufs_and_sems[2 * NBUF :]
    wid = lax.axis_index("s") * NC + lax.axis_index("c")
    base = wid * b_per_w
    pltpu.sync_copy(idx_hbm.at[pl.ds(base, b_per_w)], idx_v)

    def gather(off, b):
      pltpu.async_copy(table_hbm.at[idx_v.at[pl.ds(off, K)]], bufs[b], gsem[b])

    def gwait(b):
      pltpu.make_async_copy(table_hbm.at[pl.ds(0, K)], bufs[b], gsem[b]).wait()

    def wstart(off, b):
      pltpu.async_copy(bufs[b], out_hbm.at[pl.ds(base + off, K)], wsem[b])

    def wwait(b):
      pltpu.make_async_copy(
          table_hbm.at[pl.ds(0, K)], bufs[b], wsem[b]
      ).wait()

    for b in range(NBUF):
      gather(b * K, b)

    def body(i, carry):
      ch0 = NBUF * i
      for b in range(NBUF):
        gwait(b)
        wstart((ch0 + b) * K, b)
      for b in range(NBUF):
        wwait(b)
        nxt = jnp.minimum(ch0 + b + NBUF, n_chunks - NBUF + b) * K
        gather(nxt, b)
      return carry

    lax.fori_loop(0, n_chunks // NBUF, body, 0)
    for b in range(NBUF):
      gwait(b)

  return k


def kernel(x, table):
  V, D = table.shape
  B = x.size
  flat_idx = x.reshape((B,)).astype(jnp.int32)
  out = _make_gather(V, D, B)(flat_idx, table)
  return out.reshape(x.shape + (D,))
